# baseline (speedup 1.0000x reference)
import functools

import jax
import jax.numpy as jnp
from jax import lax
from jax.experimental import pallas as pl
from jax.experimental.pallas import tpu as pltpu
from jax.experimental.pallas import tpu_sc as plsc

NC = 2
NS = 16
NW = NC * NS
B = 128


def _celu(x):
    return jnp.where(x > 0, x, jnp.exp(jnp.minimum(x, 0.0)) - 1.0)


def _mesh():
    return plsc.VectorSubcoreMesh(core_axis_name="c", subcore_axis_name="s")


def _make_deg_kernel(n, nb):
    nzt = n // 1000

    @functools.partial(
        pl.kernel,
        out_type=[jax.ShapeDtypeStruct((n,), jnp.float32)] * 4,
        mesh=_mesh(),
        scratch_types=[
            pltpu.VMEM((nb, B), jnp.int32),
            pltpu.VMEM((nb, B), jnp.int32),
            pltpu.VMEM((nb, B), jnp.float32),
            pltpu.VMEM((1024,), jnp.float32),
            pltpu.VMEM((n,), jnp.float32),
            pltpu.VMEM_SHARED((n,), jnp.float32),
            pltpu.VMEM_SHARED((n,), jnp.float32),
            pltpu.SemaphoreType.DMA,
        ],
    )
    def deg_kernel(row_hbm, col_hbm, ew_hbm,
                   dg0_hbm, dg1_hbm, dc0_hbm, dc1_hbm,
                   row_v, col_v, ew_v, zbuf, rbuf, dg_sh, dc_sh, sem):
        c = lax.axis_index("c")
        s = lax.axis_index("s")
        wid = s * NC + c
        pltpu.sync_copy(row_hbm.at[wid], row_v)
        pltpu.sync_copy(col_hbm.at[wid], col_v)
        pltpu.sync_copy(ew_hbm.at[wid], ew_v)

        @pl.when(s < nzt)
        def _zero():
            def zb(i, carry):
                zbuf[pl.ds(i * 16, 16)] = jnp.zeros((16,), jnp.float32)
                return carry

            lax.fori_loop(0, 64, zb, 0)
            sl = pl.ds(s * 1000, 1000)
            pltpu.sync_copy(zbuf.at[pl.ds(0, 1000)], dg_sh.at[sl])
            pltpu.sync_copy(zbuf.at[pl.ds(0, 1000)], dc_sh.at[sl])

        plsc.subcore_barrier()

        def body(b, carry):
            pltpu.async_copy(ew_v.at[b], dg_sh.at[col_v.at[b]], sem,
                             add=True).wait()
            pltpu.async_copy(ew_v.at[b], dc_sh.at[row_v.at[b]], sem,
                             add=True).wait()
            return carry

        lax.fori_loop(0, nb, body, 0)
        plsc.subcore_barrier()

        @pl.when(s == 0)
        def _readback():
            pltpu.sync_copy(dg_sh, rbuf)

            @pl.when(c == 0)
            def _g0():
                pltpu.sync_copy(rbuf, dg0_hbm)

            @pl.when(c == 1)
            def _g1():
                pltpu.sync_copy(rbuf, dg1_hbm)

        @pl.when(s == 1)
        def _readback2():
            pltpu.sync_copy(dc_sh, rbuf)

            @pl.when(c == 0)
            def _c0():
                pltpu.sync_copy(rbuf, dc0_hbm)

            @pl.when(c == 1)
            def _c1():
                pltpu.sync_copy(rbuf, dc1_hbm)

    return deg_kernel


def _make_prop_kernel(n, d, nb):
    hd = d // 2
    nct = 10
    rpt = n // nct
    nf = hd // 16

    @functools.partial(
        pl.kernel,
        out_type=jax.ShapeDtypeStruct((NC, n, hd), jnp.float32),
        mesh=_mesh(),
        compiler_params=pltpu.CompilerParams(use_tc_tiling_on_sc=False),
        scratch_types=[
            pltpu.VMEM((nb, B), jnp.int32),
            pltpu.VMEM((nb, B), jnp.int32),
            pltpu.VMEM((nb, B), jnp.float32),
            pltpu.VMEM((B, 64), jnp.float32),
            pltpu.VMEM((B, 64), jnp.float32),
            pltpu.VMEM_SHARED((n, 64), jnp.float32),
            pltpu.SemaphoreType.DMA,
            pltpu.SemaphoreType.DMA,
            pltpu.SemaphoreType.DMA,
            pltpu.SemaphoreType.DMA,
        ],
    )
    def prop_kernel(u_hbm, row_hbm, col_hbm, ew_hbm, z2_hbm, out_hbm,
                    row_v, col_v, ew_v, buf0, buf1, acc_sh,
                    gsem0, gsem1, ssem0, ssem1):
        c = lax.axis_index("c")
        s = lax.axis_index("s")
        pltpu.sync_copy(row_hbm.at[c, s], row_v)
        pltpu.sync_copy(col_hbm.at[s], col_v)
        pltpu.sync_copy(ew_hbm.at[s], ew_v)

        @pl.when(s < nct)
        def _zero():
            sl = pl.ds(s * rpt, rpt)
            pltpu.sync_copy(z2_hbm.at[sl], acc_sh.at[sl])

        plsc.subcore_barrier()

        def gather(b, buf, sem):
            pltpu.async_copy(u_hbm.at[row_v.at[b]], buf, sem)

        def wait_gather(b, buf, sem):
            pltpu.make_async_copy(u_hbm.at[row_v.at[b]], buf, sem).wait()

        def scatter(b, buf, sem):
            pltpu.async_copy(buf, acc_sh.at[col_v.at[b]], sem, add=True)

        def wait_scatter(b, buf, sem):
            pltpu.make_async_copy(buf, acc_sh.at[col_v.at[b]], sem).wait()

        def scale(b, buf):
            def grp(g, carry2):
                w16 = ew_v[b, pl.ds(g * 16, 16)]
                for k in range(16):
                    j = g * 16 + k
                    w = w16[k]
                    for f in range(nf):
                        fs = pl.ds(f * 16, 16)
                        buf[j, fs] = buf[j, fs] * w
                return carry2

            lax.fori_loop(0, B // 16, grp, 0)

        bufs = (buf0, buf1)
        gsems = (gsem0, gsem1)
        ssems = (ssem0, ssem1)

        def pair(k, carry):
            for j in range(2):
                gather(2 * k + j, bufs[j], gsems[j])
            for j in range(2):
                b = 2 * k + j
                wait_gather(b, bufs[j], gsems[j])
                scale(b, bufs[j])
                scatter(b, bufs[j], ssems[j])
            for j in range(2):
                wait_scatter(2 * k + j, bufs[j], ssems[j])
            return carry

        lax.fori_loop(0, nb // 2, pair, 0)
        plsc.subcore_barrier()

        @pl.when(s < nct)
        def _readback():
            sl = pl.ds(s * rpt, rpt)
            pltpu.sync_copy(acc_sh.at[sl], out_hbm.at[c, sl])

    return prop_kernel


def _sds(shape):
    return jax.ShapeDtypeStruct(shape, jnp.float32)


def _cat(sp_ref):
    return jnp.concatenate([sp_ref[0], sp_ref[1]], axis=1)


def _tc_prep(dg0, dg1, dc0, dc1, x, w1):
    n, d = x.shape

    def body(dg0_ref, dg1_ref, dc0_ref, dc1_ref, x_ref, w_ref,
             dg_ref, dc_ref, xw_ref, u_ref):
        deg_g = dg0_ref[...] + dg1_ref[...] + 1.0
        deg_c = dc0_ref[...] + dc1_ref[...]
        dinv_g = jnp.where(deg_g > 0,
                           lax.rsqrt(jnp.where(deg_g > 0, deg_g, 1.0)), 0.0)
        dinv_c = jnp.where(deg_c > 0,
                           lax.rsqrt(jnp.where(deg_c > 0, deg_c, 1.0)), 0.0)
        dg_ref[...] = dinv_g
        dc_ref[...] = dinv_c
        xw = jnp.dot(x_ref[...], w_ref[...],
                     preferred_element_type=jnp.float32)
        xw_ref[...] = xw
        u = xw * dinv_g
        u_ref[0] = u[:, : d // 2]
        u_ref[1] = u[:, d // 2:]

    return pl.pallas_call(
        body,
        out_shape=[_sds((n, 1)), _sds((n, 1)), _sds((n, d)),
                   _sds((2, n, d // 2))],
    )(dg0.reshape(n, 1), dg1.reshape(n, 1), dc0.reshape(n, 1),
      dc1.reshape(n, 1), x, w1)


def _tc_post1(sp, xw, dinv_g, b, w_next):
    n, d = xw.shape

    def body(sp_ref, xw_ref, dg_ref, b_ref, w_ref, xw2_ref, u2_ref):
        dg = dg_ref[...]
        h = _celu(dg * _cat(sp_ref) + dg * dg * xw_ref[...] + b_ref[...])
        xw2 = jnp.dot(h, w_ref[...], preferred_element_type=jnp.float32)
        xw2_ref[...] = xw2
        u2 = xw2 * dg
        u2_ref[0] = u2[:, : d // 2]
        u2_ref[1] = u2[:, d // 2:]

    return pl.pallas_call(
        body, out_shape=[_sds((n, d)), _sds((2, n, d // 2))],
    )(sp, xw, dinv_g, b, w_next)


def _tc_post2(sp, xw, dinv_g, b, dinv_c):
    n, d = xw.shape

    def body(sp_ref, xw_ref, dg_ref, b_ref, dc_ref, h2_ref, v1_ref):
        dg = dg_ref[...]
        h2 = _celu(dg * _cat(sp_ref) + dg * dg * xw_ref[...] + b_ref[...])
        h2_ref[...] = h2
        v1 = h2 * dc_ref[...]
        v1_ref[0] = v1[:, : d // 2]
        v1_ref[1] = v1[:, d // 2:]

    return pl.pallas_call(
        body, out_shape=[_sds((n, d)), _sds((2, n, d // 2))],
    )(sp, xw, dinv_g, b, dinv_c)


def _tc_chebmid(tp, dinv_c, n, d):
    def body(tp_ref, dc_ref, tx1_ref, v2_ref):
        dc = dc_ref[...]
        tx1 = -(dc * _cat(tp_ref))
        tx1_ref[...] = tx1
        v2 = tx1 * dc
        v2_ref[0] = v2[:, : d // 2]
        v2_ref[1] = v2[:, d // 2:]

    return pl.pallas_call(
        body, out_shape=[_sds((n, d)), _sds((2, n, d // 2))],
    )(tp, dinv_c)


def _tc_final(tp, h2, tx1, dinv_c, wc, bc):
    n, d = h2.shape

    def body(tp_ref, h2_ref, tx1_ref, dc_ref, wc_ref, bc_ref, out_ref):
        h2v = h2_ref[...]
        tx2 = -2.0 * (dc_ref[...] * _cat(tp_ref)) - h2v
        out = (jnp.dot(h2v, wc_ref[0], preferred_element_type=jnp.float32)
               + jnp.dot(tx1_ref[...], wc_ref[1],
                         preferred_element_type=jnp.float32)
               + jnp.dot(tx2, wc_ref[2], preferred_element_type=jnp.float32)
               + bc_ref[...])
        out_ref[...] = _celu(out)

    return pl.pallas_call(
        body, out_shape=_sds((n, d)),
    )(tp, h2, tx1, dinv_c, wc, bc)


def kernel(x, edge_index, edge_weight, W1, b1, W2, b2, Wc, bc):
    n, d = x.shape
    e = edge_weight.shape[0]
    row = edge_index[0].astype(jnp.int32)
    col = edge_index[1].astype(jnp.int32)
    ew = edge_weight.astype(jnp.float32)

    def padded(a, nchunk, nbatch, fill):
        padn = nchunk * nbatch * B - e
        a = jnp.concatenate([a, jnp.full((padn,), fill, a.dtype)])
        return a.reshape(nchunk, nbatch, B)

    nb32 = -(-e // (NW * B))
    row32 = padded(row, NW, nb32, 0)
    col32 = padded(col, NW, nb32, 0)
    ew32 = padded(ew, NW, nb32, 0.0)

    nb16 = -(-e // (NS * B))
    nb16 += nb16 % 2
    row16 = padded(row, NS, nb16, 0)
    rowc = jnp.stack([row16, row16 + n])
    col16 = padded(col, NS, nb16, 0)
    ew16 = padded(ew, NS, nb16, 0.0)
    zh = jnp.zeros((n, d // 2), jnp.float32)

    dg0, dg1, dc0, dc1 = _make_deg_kernel(n, nb32)(row32, col32, ew32)
    dinv_g, dinv_c, xw1, u1 = _tc_prep(dg0, dg1, dc0, dc1, x, W1)

    prop = _make_prop_kernel(n, d, nb16)

    def run_prop(u_halves):
        return prop(u_halves.reshape(2 * n, d // 2), rowc, col16, ew16, zh)

    s1 = run_prop(u1)
    xw2, u2 = _tc_post1(s1, xw1, dinv_g, b1.reshape(1, d), W2)
    s2 = run_prop(u2)
    h2, v1 = _tc_post2(s2, xw2, dinv_g, b2.reshape(1, d), dinv_c)
    t1 = run_prop(v1)
    tx1, v2 = _tc_chebmid(t1, dinv_c, n, d)
    t2 = run_prop(v2)
    return _tc_final(t2, h2, tx1, dinv_c, Wc, bc)


# speedup vs baseline: 1.3177x; 1.3177x over previous
import functools

import jax
import jax.numpy as jnp
from jax import lax
from jax.experimental import pallas as pl
from jax.experimental.pallas import tpu as pltpu
from jax.experimental.pallas import tpu_sc as plsc

NC = 2
NS = 16
NW = NC * NS
B = 128


def _celu(x):
    return jnp.where(x > 0, x, jnp.exp(jnp.minimum(x, 0.0)) - 1.0)


def _mesh():
    return plsc.VectorSubcoreMesh(core_axis_name="c", subcore_axis_name="s")


def _make_deg_kernel(n, nb):
    nzt = n // 1000

    @functools.partial(
        pl.kernel,
        out_type=[jax.ShapeDtypeStruct((n,), jnp.float32)] * 4,
        mesh=_mesh(),
        scratch_types=[
            pltpu.VMEM((nb, B), jnp.int32),
            pltpu.VMEM((nb, B), jnp.int32),
            pltpu.VMEM((nb, B), jnp.float32),
            pltpu.VMEM((1024,), jnp.float32),
            pltpu.VMEM((n,), jnp.float32),
            pltpu.VMEM_SHARED((n,), jnp.float32),
            pltpu.VMEM_SHARED((n,), jnp.float32),
            pltpu.SemaphoreType.DMA,
        ],
    )
    def deg_kernel(row_hbm, col_hbm, ew_hbm,
                   dg0_hbm, dg1_hbm, dc0_hbm, dc1_hbm,
                   row_v, col_v, ew_v, zbuf, rbuf, dg_sh, dc_sh, sem):
        c = lax.axis_index("c")
        s = lax.axis_index("s")
        wid = s * NC + c
        pltpu.sync_copy(row_hbm.at[wid], row_v)
        pltpu.sync_copy(col_hbm.at[wid], col_v)
        pltpu.sync_copy(ew_hbm.at[wid], ew_v)

        @pl.when(s < nzt)
        def _zero():
            def zb(i, carry):
                zbuf[pl.ds(i * 16, 16)] = jnp.zeros((16,), jnp.float32)
                return carry

            lax.fori_loop(0, 64, zb, 0)
            sl = pl.ds(s * 1000, 1000)
            pltpu.sync_copy(zbuf.at[pl.ds(0, 1000)], dg_sh.at[sl])
            pltpu.sync_copy(zbuf.at[pl.ds(0, 1000)], dc_sh.at[sl])

        plsc.subcore_barrier()

        def body(b, carry):
            pltpu.async_copy(ew_v.at[b], dg_sh.at[col_v.at[b]], sem,
                             add=True).wait()
            pltpu.async_copy(ew_v.at[b], dc_sh.at[row_v.at[b]], sem,
                             add=True).wait()
            return carry

        lax.fori_loop(0, nb, body, 0)
        plsc.subcore_barrier()

        @pl.when(s == 0)
        def _readback():
            pltpu.sync_copy(dg_sh, rbuf)

            @pl.when(c == 0)
            def _g0():
                pltpu.sync_copy(rbuf, dg0_hbm)

            @pl.when(c == 1)
            def _g1():
                pltpu.sync_copy(rbuf, dg1_hbm)

        @pl.when(s == 1)
        def _readback2():
            pltpu.sync_copy(dc_sh, rbuf)

            @pl.when(c == 0)
            def _c0():
                pltpu.sync_copy(rbuf, dc0_hbm)

            @pl.when(c == 1)
            def _c1():
                pltpu.sync_copy(rbuf, dc1_hbm)

    return deg_kernel


def _make_prop_kernel(n, d, nb):
    nct = 10
    rpt = n // nct
    nf = d // 16

    @functools.partial(
        pl.kernel,
        out_type=jax.ShapeDtypeStruct((NC, n, d), jnp.float32),
        mesh=_mesh(),
        scratch_types=[
            pltpu.VMEM((nb, B), jnp.int32),
            pltpu.VMEM((nb, B), jnp.int32),
            pltpu.VMEM((nb, B), jnp.float32),
            pltpu.VMEM((B, d), jnp.float32),
            pltpu.VMEM_SHARED((n, d), jnp.float32),
            pltpu.SemaphoreType.DMA,
            pltpu.SemaphoreType.DMA,
            pltpu.SemaphoreType.DMA,
        ],
    )
    def prop_kernel(u_hbm, row_hbm, col_hbm, ew_hbm, z2_hbm, out_hbm,
                    row_v, col_v, ew_v, rows_v, acc_sh, gsem0, gsem1, ssem):
        c = lax.axis_index("c")
        s = lax.axis_index("s")
        wid = s * NC + c
        pltpu.sync_copy(row_hbm.at[wid], row_v)
        pltpu.sync_copy(col_hbm.at[wid], col_v)
        pltpu.sync_copy(ew_hbm.at[wid], ew_v)

        @pl.when(s < nct)
        def _zero():
            sl = pl.ds(s * rpt, rpt)
            pltpu.sync_copy(z2_hbm.at[sl], acc_sh.at[sl])

        plsc.subcore_barrier()

        hb = B // 2

        def scale_half(b, lo):
            def grp(g, carry2):
                w16 = ew_v[b, pl.ds(lo + g * 16, 16)]
                for k in range(16):
                    j = lo + g * 16 + k
                    w = w16[k]
                    for f in range(nf):
                        fs = pl.ds(f * 16, 16)
                        rows_v[j, fs] = rows_v[j, fs] * w
                return carry2

            lax.fori_loop(0, hb // 16, grp, 0)

        def body(b, carry):
            g0 = pltpu.async_copy(
                u_hbm.at[row_v.at[b, pl.ds(0, hb)]],
                rows_v.at[pl.ds(0, hb)], gsem0)
            g1 = pltpu.async_copy(
                u_hbm.at[row_v.at[b, pl.ds(hb, hb)]],
                rows_v.at[pl.ds(hb, hb)], gsem1)
            g0.wait()
            scale_half(b, 0)
            g1.wait()
            scale_half(b, hb)
            pltpu.async_copy(rows_v, acc_sh.at[col_v.at[b]], ssem,
                             add=True).wait()
            return carry

        lax.fori_loop(0, nb, body, 0)
        plsc.subcore_barrier()

        @pl.when(s < nct)
        def _readback():
            sl = pl.ds(s * rpt, rpt)
            pltpu.sync_copy(acc_sh.at[sl], out_hbm.at[c, sl])

    return prop_kernel


def _sds(shape):
    return jax.ShapeDtypeStruct(shape, jnp.float32)


def _tc_prep(dg0, dg1, dc0, dc1, x, w1):
    n, d = x.shape

    def body(dg0_ref, dg1_ref, dc0_ref, dc1_ref, x_ref, w_ref,
             dg_ref, dc_ref, xw_ref, u_ref):
        deg_g = dg0_ref[...] + dg1_ref[...] + 1.0
        deg_c = dc0_ref[...] + dc1_ref[...]
        dinv_g = jnp.where(deg_g > 0,
                           lax.rsqrt(jnp.where(deg_g > 0, deg_g, 1.0)), 0.0)
        dinv_c = jnp.where(deg_c > 0,
                           lax.rsqrt(jnp.where(deg_c > 0, deg_c, 1.0)), 0.0)
        dg_ref[...] = dinv_g
        dc_ref[...] = dinv_c
        xw = jnp.dot(x_ref[...], w_ref[...],
                     preferred_element_type=jnp.float32)
        xw_ref[...] = xw
        u_ref[...] = xw * dinv_g

    return pl.pallas_call(
        body,
        out_shape=[_sds((n, 1)), _sds((n, 1)), _sds((n, d)), _sds((n, d))],
    )(dg0.reshape(n, 1), dg1.reshape(n, 1), dc0.reshape(n, 1),
      dc1.reshape(n, 1), x, w1)


def _tc_post1(spart, xw, dinv_g, b, w_next):
    n, d = xw.shape

    def body(sp_ref, xw_ref, dg_ref, b_ref, w_ref, xw2_ref, u2_ref):
        dg = dg_ref[...]
        h = _celu(dg * (sp_ref[0] + sp_ref[1]) + dg * dg * xw_ref[...]
                  + b_ref[...])
        xw2 = jnp.dot(h, w_ref[...], preferred_element_type=jnp.float32)
        xw2_ref[...] = xw2
        u2_ref[...] = xw2 * dg

    return pl.pallas_call(
        body, out_shape=[_sds((n, d)), _sds((n, d))],
    )(spart, xw, dinv_g, b, w_next)


def _tc_post2(spart, xw, dinv_g, b, dinv_c):
    n, d = xw.shape

    def body(sp_ref, xw_ref, dg_ref, b_ref, dc_ref, h2_ref, v1_ref):
        dg = dg_ref[...]
        h2 = _celu(dg * (sp_ref[0] + sp_ref[1]) + dg * dg * xw_ref[...]
                   + b_ref[...])
        h2_ref[...] = h2
        v1_ref[...] = h2 * dc_ref[...]

    return pl.pallas_call(
        body, out_shape=[_sds((n, d)), _sds((n, d))],
    )(spart, xw, dinv_g, b, dinv_c)


def _tc_chebmid(tpart, dinv_c):
    _, n, d = tpart.shape

    def body(tp_ref, dc_ref, tx1_ref, v2_ref):
        dc = dc_ref[...]
        tx1 = -(dc * (tp_ref[0] + tp_ref[1]))
        tx1_ref[...] = tx1
        v2_ref[...] = tx1 * dc

    return pl.pallas_call(
        body, out_shape=[_sds((n, d)), _sds((n, d))],
    )(tpart, dinv_c)


def _tc_final(tpart, h2, tx1, dinv_c, wc, bc):
    n, d = h2.shape

    def body(tp_ref, h2_ref, tx1_ref, dc_ref, wc_ref, bc_ref, out_ref):
        h2v = h2_ref[...]
        tx2 = -2.0 * (dc_ref[...] * (tp_ref[0] + tp_ref[1])) - h2v
        out = (jnp.dot(h2v, wc_ref[0], preferred_element_type=jnp.float32)
               + jnp.dot(tx1_ref[...], wc_ref[1],
                         preferred_element_type=jnp.float32)
               + jnp.dot(tx2, wc_ref[2], preferred_element_type=jnp.float32)
               + bc_ref[...])
        out_ref[...] = _celu(out)

    return pl.pallas_call(
        body, out_shape=_sds((n, d)),
    )(tpart, h2, tx1, dinv_c, wc, bc)


def kernel(x, edge_index, edge_weight, W1, b1, W2, b2, Wc, bc):
    n, d = x.shape
    e = edge_weight.shape[0]
    nb = -(-e // (NW * B))
    ep = nb * B * NW
    pad = ep - e

    row = edge_index[0].astype(jnp.int32)
    col = edge_index[1].astype(jnp.int32)
    ew = edge_weight.astype(jnp.float32)
    if pad:
        row = jnp.concatenate([row, jnp.zeros((pad,), jnp.int32)])
        col = jnp.concatenate([col, jnp.zeros((pad,), jnp.int32)])
        ew = jnp.concatenate([ew, jnp.zeros((pad,), jnp.float32)])
    row3 = row.reshape(NW, nb, B)
    col3 = col.reshape(NW, nb, B)
    ew3 = ew.reshape(NW, nb, B)
    z1 = jnp.zeros((n,), jnp.float32)
    z2 = jnp.zeros((n, d), jnp.float32)

    dg0, dg1, dc0, dc1 = _make_deg_kernel(n, nb)(row3, col3, ew3)
    dinv_g, dinv_c, xw1, u1 = _tc_prep(dg0, dg1, dc0, dc1, x, W1)

    prop = _make_prop_kernel(n, d, nb)
    s1 = prop(u1, row3, col3, ew3, z2)
    xw2, u2 = _tc_post1(s1, xw1, dinv_g, b1.reshape(1, d), W2)
    s2 = prop(u2, row3, col3, ew3, z2)
    h2, v1 = _tc_post2(s2, xw2, dinv_g, b2.reshape(1, d), dinv_c)
    t1 = prop(v1, row3, col3, ew3, z2)
    tx1, v2 = _tc_chebmid(t1, dinv_c)
    t2 = prop(v2, row3, col3, ew3, z2)
    return _tc_final(t2, h2, tx1, dinv_c, Wc, bc)
